# baseline (speedup 1.0000x reference)
import functools

import jax
import jax.numpy as jnp
from jax import lax
from jax.experimental import pallas as pl
from jax.experimental.pallas import tpu as pltpu
from jax.experimental.pallas import tpu_sc as plsc

_VOCAB = 1000000
_D = 32
_B = 4096
_L = 200

_INFO = plsc.get_sparse_core_info()
_NC = _INFO.num_cores
_NS = _INFO.num_subcores
_NW = _NC * _NS
_BPW = _B // _NW


def _sc_pool_body(ids_hbm, table_hbm, out_hbm, idx_v, acc_v, sem):
    c = lax.axis_index("c")
    s = lax.axis_index("s")
    wid = s * _NC + c
    base = wid * _BPW

    pltpu.sync_copy(ids_hbm.at[:, pl.ds(base, _BPW)], idx_v)

    def zbody(i, carry):
        zero = jnp.zeros((16,), jnp.float32)
        acc_v[i, pl.ds(0, 16)] = zero
        acc_v[i, pl.ds(16, 16)] = zero
        return carry

    lax.fori_loop(0, _BPW, zbody, 0)

    def gbody(j, carry):
        pltpu.async_copy(table_hbm.at[idx_v.at[j]], acc_v, sem, add=True)
        return carry

    lax.fori_loop(0, _L, gbody, 0)

    def wbody(j, carry):
        pltpu.make_async_copy(table_hbm.at[idx_v.at[0]], acc_v, sem).wait()
        return carry

    lax.fori_loop(0, _L, wbody, 0)

    pltpu.sync_copy(acc_v, out_hbm.at[pl.ds(base, _BPW), :])


@jax.jit
def _sc_pool(ids_t, table):
    mesh = plsc.VectorSubcoreMesh(core_axis_name="c", subcore_axis_name="s")
    f = pl.kernel(
        _sc_pool_body,
        out_type=jax.ShapeDtypeStruct((_B, _D), jnp.float32),
        mesh=mesh,
        scratch_types=[
            pltpu.VMEM((_L, _BPW), jnp.int32),
            pltpu.VMEM((_BPW, _D), jnp.float32),
            pltpu.SemaphoreType.DMA,
        ],
        compiler_params=pltpu.CompilerParams(use_tc_tiling_on_sc=False),
    )
    return f(ids_t, table)


def _tc_detile_body(x_ref, o_ref):
    o_ref[...] = x_ref[...].reshape(-1)


@jax.jit
def _tc_detile(ids_t):
    return pl.pallas_call(
        _tc_detile_body,
        out_shape=jax.ShapeDtypeStruct((_L * _B,), jnp.int32),
    )(ids_t)


_NCT = _VOCAB // 128
_TAIL = _VOCAB - _NCT * 128


def _sc_transpose_body(tt_hbm, tail_hbm, out_hbm, slab_v, obuf_v, tbuf_v,
                       sem_in, sem_out):
    c = lax.axis_index("c")
    s = lax.axis_index("s")
    wid = s * _NC + c
    n_lo = _NCT // _NW
    n_extra = _NCT - n_lo * _NW
    start = jnp.where(wid < n_extra, wid * (n_lo + 1),
                      n_extra * (n_lo + 1) + (wid - n_extra) * n_lo)
    count = jnp.where(wid < n_extra, n_lo + 1, n_lo)

    d_lo = lax.iota(jnp.int32, 16)
    d_hi = d_lo + 16

    def transpose_slab(buf_idx, nv):
        pp = jnp.full((16,), buf_idx, jnp.int32)

        def vbody(v, carry):
            vv = jnp.full((16,), v, jnp.int32)
            g0 = plsc.load_gather(slab_v, [pp, d_lo, vv])
            g1 = plsc.load_gather(slab_v, [pp, d_hi, vv])
            obuf_v[buf_idx, pl.ds(v * _D, 16)] = g0
            obuf_v[buf_idx, pl.ds(v * _D + 16, 16)] = g1
            return carry

        lax.fori_loop(0, nv, vbody, 0)

    pltpu.async_copy(tt_hbm.at[:, pl.ds(start * 128, 128)],
                     slab_v.at[0], sem_in)

    def body(k, carry):
        p = lax.rem(k, 2)

        @pl.when(k + 1 < count)
        def _prefetch():
            pltpu.async_copy(
                tt_hbm.at[:, pl.ds((start + k + 1) * 128, 128)],
                slab_v.at[lax.rem(k + 1, 2)], sem_in)

        pltpu.make_async_copy(tt_hbm.at[:, pl.ds(0, 128)],
                              slab_v.at[0], sem_in).wait()

        @pl.when(k >= 2)
        def _free_obuf():
            pltpu.make_async_copy(obuf_v.at[0],
                                  out_hbm.at[pl.ds(0, 128 * _D)],
                                  sem_out).wait()

        transpose_slab(p, 128)
        pltpu.async_copy(obuf_v.at[p],
                         out_hbm.at[pl.ds((start + k) * 128 * _D, 128 * _D)],
                         sem_out)
        return carry

    lax.fori_loop(0, count, body, 0)

    def drain(j, carry):
        pltpu.make_async_copy(obuf_v.at[0], out_hbm.at[pl.ds(0, 128 * _D)],
                              sem_out).wait()
        return carry

    lax.fori_loop(0, jnp.minimum(count, 2), drain, 0)

    @pl.when(wid == _NW - 1)
    def _tail():
        pltpu.sync_copy(tail_hbm, tbuf_v)
        pltpu.sync_copy(tbuf_v,
                        out_hbm.at[pl.ds(_NCT * 128 * _D, _TAIL * _D)])


@jax.jit
def _sc_transpose(tt, tail1d):
    mesh = plsc.VectorSubcoreMesh(core_axis_name="c", subcore_axis_name="s")
    f = pl.kernel(
        _sc_transpose_body,
        out_type=jax.ShapeDtypeStruct((_VOCAB * _D,), jnp.float32),
        mesh=mesh,
        scratch_types=[
            pltpu.VMEM((2, _D, 128), jnp.float32),
            pltpu.VMEM((2, 128 * _D), jnp.float32),
            pltpu.VMEM((_TAIL * _D,), jnp.float32),
            pltpu.SemaphoreType.DMA,
            pltpu.SemaphoreType.DMA,
        ],
        compiler_params=pltpu.CompilerParams(
            use_tc_tiling_on_sc=True, needs_layout_passes=False),
    )
    return f(tt, tail1d).reshape(_VOCAB, _D)


def _tc_linear_body(x_ref, wt_ref, b_ref, o_ref):
    o_ref[...] = (
        jnp.dot(x_ref[...], wt_ref[...], preferred_element_type=jnp.float32)
        + b_ref[...]
    )


@jax.jit
def _tc_linear(sums, wt_scaled, b2d):
    return pl.pallas_call(
        _tc_linear_body,
        out_shape=jax.ShapeDtypeStruct((_B, 2), jnp.float32),
    )(sums, wt_scaled, b2d)


def kernel(input_ids, embedding, W, b):
    ids_t = input_ids.T.astype(jnp.int32)
    ids_lin = _tc_detile(ids_t).reshape(_L, _B)
    tail1d = embedding[_NCT * 128:, :].reshape(-1)
    table_lin = _sc_transpose(embedding.T, tail1d)
    sums = _sc_pool(ids_lin, table_lin)
    wt_scaled = (W.T / jnp.float32(_L)).astype(jnp.float32)
    b2d = b.reshape(1, 2).astype(jnp.float32)
    return _tc_linear(sums, wt_scaled, b2d)


# speedup vs baseline: 1.3220x; 1.3220x over previous
import functools

import jax
import jax.numpy as jnp
from jax import lax
from jax.experimental import pallas as pl
from jax.experimental.pallas import tpu as pltpu
from jax.experimental.pallas import tpu_sc as plsc

_VOCAB = 1000000
_D = 32
_B = 4096
_L = 200

_INFO = plsc.get_sparse_core_info()
_NC = _INFO.num_cores
_NS = _INFO.num_subcores
_NW = _NC * _NS
_BPW = _B // _NW


def _sc_pool_body(ids_hbm, table_hbm, out_hbm, idx_v, acc_v, sem):
    c = lax.axis_index("c")
    s = lax.axis_index("s")
    wid = s * _NC + c
    base = wid * _BPW

    pltpu.sync_copy(ids_hbm.at[:, pl.ds(base, _BPW)], idx_v)

    def zbody(i, carry):
        zero = jnp.zeros((16,), jnp.float32)
        acc_v[i, pl.ds(0, 16)] = zero
        acc_v[i, pl.ds(16, 16)] = zero
        return carry

    lax.fori_loop(0, _BPW, zbody, 0)

    def gbody(j, carry):
        pltpu.async_copy(table_hbm.at[idx_v.at[j]], acc_v, sem, add=True)
        return carry

    lax.fori_loop(0, _L, gbody, 0)

    def wbody(j, carry):
        pltpu.make_async_copy(table_hbm.at[idx_v.at[0]], acc_v, sem).wait()
        return carry

    lax.fori_loop(0, _L, wbody, 0)

    pltpu.sync_copy(acc_v, out_hbm.at[pl.ds(base, _BPW), :])


@jax.jit
def _sc_pool(ids_t, table):
    mesh = plsc.VectorSubcoreMesh(core_axis_name="c", subcore_axis_name="s")
    f = pl.kernel(
        _sc_pool_body,
        out_type=jax.ShapeDtypeStruct((_B, _D), jnp.float32),
        mesh=mesh,
        scratch_types=[
            pltpu.VMEM((_L, _BPW), jnp.int32),
            pltpu.VMEM((_BPW, _D), jnp.float32),
            pltpu.SemaphoreType.DMA,
        ],
        compiler_params=pltpu.CompilerParams(use_tc_tiling_on_sc=False),
    )
    return f(ids_t, table)


def _tc_detile_body(x_ref, o_ref):
    o_ref[...] = x_ref[...].reshape(-1)


@jax.jit
def _tc_detile(ids_t):
    return pl.pallas_call(
        _tc_detile_body,
        out_shape=jax.ShapeDtypeStruct((_L * _B,), jnp.int32),
    )(ids_t)


_NCT = _VOCAB // 128
_TAIL = _VOCAB - _NCT * 128


def _sc_transpose_body(tt_hbm, tail_hbm, out_hbm, slab_v, obuf_v, tbuf_v,
                       sem_in, sem_out):
    c = lax.axis_index("c")
    s = lax.axis_index("s")
    wid = s * _NC + c
    n_lo = _NCT // _NW
    n_extra = _NCT - n_lo * _NW
    start = jnp.where(wid < n_extra, wid * (n_lo + 1),
                      n_extra * (n_lo + 1) + (wid - n_extra) * n_lo)
    count = jnp.where(wid < n_extra, n_lo + 1, n_lo)

    d_lo = lax.iota(jnp.int32, 16)
    d_hi = d_lo + 16

    def transpose_slab(buf_idx, nv):
        pp = jnp.full((16,), buf_idx, jnp.int32)

        @plsc.parallel_loop(0, nv, step=1, unroll=16)
        def vbody(v):
            vv = jnp.full((16,), v, jnp.int32)
            g0 = plsc.load_gather(slab_v, [pp, d_lo, vv])
            g1 = plsc.load_gather(slab_v, [pp, d_hi, vv])
            obuf_v[buf_idx, pl.ds(v * _D, 16)] = g0
            obuf_v[buf_idx, pl.ds(v * _D + 16, 16)] = g1

    pltpu.async_copy(tt_hbm.at[:, pl.ds(start * 128, 128)],
                     slab_v.at[0], sem_in)

    def body(k, carry):
        p = lax.rem(k, 2)

        @pl.when(k + 1 < count)
        def _prefetch():
            pltpu.async_copy(
                tt_hbm.at[:, pl.ds((start + k + 1) * 128, 128)],
                slab_v.at[lax.rem(k + 1, 2)], sem_in)

        pltpu.make_async_copy(tt_hbm.at[:, pl.ds(0, 128)],
                              slab_v.at[0], sem_in).wait()

        @pl.when(k >= 2)
        def _free_obuf():
            pltpu.make_async_copy(obuf_v.at[0],
                                  out_hbm.at[pl.ds(0, 128 * _D)],
                                  sem_out).wait()

        transpose_slab(p, 128)
        pltpu.async_copy(obuf_v.at[p],
                         out_hbm.at[pl.ds((start + k) * 128 * _D, 128 * _D)],
                         sem_out)
        return carry

    lax.fori_loop(0, count, body, 0)

    def drain(j, carry):
        pltpu.make_async_copy(obuf_v.at[0], out_hbm.at[pl.ds(0, 128 * _D)],
                              sem_out).wait()
        return carry

    lax.fori_loop(0, jnp.minimum(count, 2), drain, 0)

    @pl.when(wid == _NW - 1)
    def _tail():
        pltpu.sync_copy(tail_hbm, tbuf_v)
        pltpu.sync_copy(tbuf_v,
                        out_hbm.at[pl.ds(_NCT * 128 * _D, _TAIL * _D)])


@jax.jit
def _sc_transpose(tt, tail1d):
    mesh = plsc.VectorSubcoreMesh(core_axis_name="c", subcore_axis_name="s")
    f = pl.kernel(
        _sc_transpose_body,
        out_type=jax.ShapeDtypeStruct((_VOCAB * _D,), jnp.float32),
        mesh=mesh,
        scratch_types=[
            pltpu.VMEM((2, _D, 128), jnp.float32),
            pltpu.VMEM((2, 128 * _D), jnp.float32),
            pltpu.VMEM((_TAIL * _D,), jnp.float32),
            pltpu.SemaphoreType.DMA,
            pltpu.SemaphoreType.DMA,
        ],
        compiler_params=pltpu.CompilerParams(
            use_tc_tiling_on_sc=True, needs_layout_passes=False),
    )
    return f(tt, tail1d).reshape(_VOCAB, _D)


def _tc_linear_body(x_ref, wt_ref, b_ref, o_ref):
    o_ref[...] = (
        jnp.dot(x_ref[...], wt_ref[...], preferred_element_type=jnp.float32)
        + b_ref[...]
    )


@jax.jit
def _tc_linear(sums, wt_scaled, b2d):
    return pl.pallas_call(
        _tc_linear_body,
        out_shape=jax.ShapeDtypeStruct((_B, 2), jnp.float32),
    )(sums, wt_scaled, b2d)


def kernel(input_ids, embedding, W, b):
    ids_t = input_ids.T.astype(jnp.int32)
    ids_lin = _tc_detile(ids_t).reshape(_L, _B)
    tail1d = embedding[_NCT * 128:, :].reshape(-1)
    table_lin = _sc_transpose(embedding.T, tail1d)
    sums = _sc_pool(ids_lin, table_lin)
    wt_scaled = (W.T / jnp.float32(_L)).astype(jnp.float32)
    b2d = b.reshape(1, 2).astype(jnp.float32)
    return _tc_linear(sums, wt_scaled, b2d)


# speedup vs baseline: 1.3263x; 1.0032x over previous
import functools

import jax
import jax.numpy as jnp
from jax import lax
from jax.experimental import pallas as pl
from jax.experimental.pallas import tpu as pltpu
from jax.experimental.pallas import tpu_sc as plsc

_VOCAB = 1000000
_D = 32
_B = 4096
_L = 200

_INFO = plsc.get_sparse_core_info()
_NC = _INFO.num_cores
_NS = _INFO.num_subcores
_NW = _NC * _NS
_BPW = _B // _NW


def _sc_pool_body(ids_hbm, table_hbm, out_hbm, idx_v, acc_v, sem):
    c = lax.axis_index("c")
    s = lax.axis_index("s")
    wid = s * _NC + c
    base = wid * _BPW

    pltpu.sync_copy(ids_hbm.at[:, pl.ds(base, _BPW)], idx_v)

    def zbody(i, carry):
        zero = jnp.zeros((16,), jnp.float32)
        acc_v[i, pl.ds(0, 16)] = zero
        acc_v[i, pl.ds(16, 16)] = zero
        return carry

    lax.fori_loop(0, _BPW, zbody, 0)

    def gbody(j, carry):
        pltpu.async_copy(table_hbm.at[idx_v.at[j]], acc_v, sem, add=True)
        return carry

    lax.fori_loop(0, _L, gbody, 0)

    def wbody(j, carry):
        pltpu.make_async_copy(table_hbm.at[idx_v.at[0]], acc_v, sem).wait()
        return carry

    lax.fori_loop(0, _L, wbody, 0)

    pltpu.sync_copy(acc_v, out_hbm.at[pl.ds(base, _BPW), :])


@jax.jit
def _sc_pool(ids_t, table):
    mesh = plsc.VectorSubcoreMesh(core_axis_name="c", subcore_axis_name="s")
    f = pl.kernel(
        _sc_pool_body,
        out_type=jax.ShapeDtypeStruct((_B, _D), jnp.float32),
        mesh=mesh,
        scratch_types=[
            pltpu.VMEM((_L, _BPW), jnp.int32),
            pltpu.VMEM((_BPW, _D), jnp.float32),
            pltpu.SemaphoreType.DMA,
        ],
        compiler_params=pltpu.CompilerParams(use_tc_tiling_on_sc=False),
    )
    return f(ids_t, table)


def _tc_detile_body(x_ref, o_ref):
    o_ref[...] = x_ref[...].reshape(-1)


@jax.jit
def _tc_detile(ids_t):
    return pl.pallas_call(
        _tc_detile_body,
        out_shape=jax.ShapeDtypeStruct((_L * _B,), jnp.int32),
    )(ids_t)


_NCT = _VOCAB // 128
_TAIL = _VOCAB - _NCT * 128


def _sc_transpose_body(tt_hbm, tail_hbm, out_hbm, slab_v, obuf_v, tbuf_v,
                       sem_in, sem_out):
    c = lax.axis_index("c")
    s = lax.axis_index("s")
    wid = s * _NC + c
    n_lo = _NCT // _NW
    n_extra = _NCT - n_lo * _NW
    start = jnp.where(wid < n_extra, wid * (n_lo + 1),
                      n_extra * (n_lo + 1) + (wid - n_extra) * n_lo)
    count = jnp.where(wid < n_extra, n_lo + 1, n_lo)

    d_lo = lax.iota(jnp.int32, 16)
    d_hi = d_lo + 16

    def transpose_slab(buf_idx, nv):
        pp = jnp.full((16,), buf_idx, jnp.int32)

        @plsc.parallel_loop(0, nv, step=1, unroll=16)
        def vbody(v):
            vv = jnp.full((16,), v, jnp.int32)
            g0 = plsc.load_gather(slab_v, [pp, d_lo, vv])
            g1 = plsc.load_gather(slab_v, [pp, d_hi, vv])
            obuf_v[buf_idx, pl.ds(v * _D, 16)] = g0
            obuf_v[buf_idx, pl.ds(v * _D + 16, 16)] = g1

    pltpu.async_copy(tt_hbm.at[:, pl.ds(start * 128, 128)],
                     slab_v.at[0, :, pl.ds(0, 128)], sem_in)

    def body(k, carry):
        p = lax.rem(k, 2)

        @pl.when(k + 1 < count)
        def _prefetch():
            pltpu.async_copy(
                tt_hbm.at[:, pl.ds((start + k + 1) * 128, 128)],
                slab_v.at[lax.rem(k + 1, 2), :, pl.ds(0, 128)], sem_in)

        pltpu.make_async_copy(tt_hbm.at[:, pl.ds(0, 128)],
                              slab_v.at[0, :, pl.ds(0, 128)], sem_in).wait()

        @pl.when(k >= 2)
        def _free_obuf():
            pltpu.make_async_copy(obuf_v.at[0],
                                  out_hbm.at[pl.ds(0, 128 * _D)],
                                  sem_out).wait()

        transpose_slab(p, 128)
        pltpu.async_copy(obuf_v.at[p],
                         out_hbm.at[pl.ds((start + k) * 128 * _D, 128 * _D)],
                         sem_out)
        return carry

    lax.fori_loop(0, count, body, 0)

    def drain(j, carry):
        pltpu.make_async_copy(obuf_v.at[0], out_hbm.at[pl.ds(0, 128 * _D)],
                              sem_out).wait()
        return carry

    lax.fori_loop(0, jnp.minimum(count, 2), drain, 0)

    @pl.when(wid == _NW - 1)
    def _tail():
        pltpu.sync_copy(tail_hbm, tbuf_v)
        pltpu.sync_copy(tbuf_v,
                        out_hbm.at[pl.ds(_NCT * 128 * _D, _TAIL * _D)])


@jax.jit
def _sc_transpose(tt, tail1d):
    mesh = plsc.VectorSubcoreMesh(core_axis_name="c", subcore_axis_name="s")
    f = pl.kernel(
        _sc_transpose_body,
        out_type=jax.ShapeDtypeStruct((_VOCAB * _D,), jnp.float32),
        mesh=mesh,
        scratch_types=[
            pltpu.VMEM((2, _D, 137), jnp.float32),
            pltpu.VMEM((2, 128 * _D), jnp.float32),
            pltpu.VMEM((_TAIL * _D,), jnp.float32),
            pltpu.SemaphoreType.DMA,
            pltpu.SemaphoreType.DMA,
        ],
        compiler_params=pltpu.CompilerParams(
            use_tc_tiling_on_sc=True, needs_layout_passes=False),
    )
    return f(tt, tail1d).reshape(_VOCAB, _D)


def _tc_linear_body(x_ref, wt_ref, b_ref, o_ref):
    o_ref[...] = (
        jnp.dot(x_ref[...], wt_ref[...], preferred_element_type=jnp.float32)
        + b_ref[...]
    )


@jax.jit
def _tc_linear(sums, wt_scaled, b2d):
    return pl.pallas_call(
        _tc_linear_body,
        out_shape=jax.ShapeDtypeStruct((_B, 2), jnp.float32),
    )(sums, wt_scaled, b2d)


def kernel(input_ids, embedding, W, b):
    ids_t = input_ids.T.astype(jnp.int32)
    ids_lin = _tc_detile(ids_t).reshape(_L, _B)
    tail1d = embedding[_NCT * 128:, :].reshape(-1)
    table_lin = _sc_transpose(embedding.T, tail1d)
    sums = _sc_pool(ids_lin, table_lin)
    wt_scaled = (W.T / jnp.float32(_L)).astype(jnp.float32)
    b2d = b.reshape(1, 2).astype(jnp.float32)
    return _tc_linear(sums, wt_scaled, b2d)


# speedup vs baseline: 4.1083x; 3.0976x over previous
import functools

import jax
import jax.numpy as jnp
from jax import lax
from jax.experimental import pallas as pl
from jax.experimental.pallas import tpu as pltpu
from jax.experimental.pallas import tpu_sc as plsc

_VOCAB = 1000000
_D = 32
_B = 4096
_L = 200

_INFO = plsc.get_sparse_core_info()
_NC = _INFO.num_cores
_NS = _INFO.num_subcores
_NW = _NC * _NS
_BPW = _B // _NW


def _sc_pool_body(ids_hbm, table_hbm, out_hbm, idx_v, acc_v, sem):
    c = lax.axis_index("c")
    s = lax.axis_index("s")
    wid = s * _NC + c
    base = wid * _BPW

    pltpu.sync_copy(ids_hbm.at[:, pl.ds(base, _BPW)], idx_v)

    def zbody(i, carry):
        zero = jnp.zeros((16,), jnp.float32)
        acc_v[i, pl.ds(0, 16)] = zero
        acc_v[i, pl.ds(16, 16)] = zero
        return carry

    lax.fori_loop(0, _BPW, zbody, 0)

    def gbody(j, carry):
        pltpu.async_copy(table_hbm.at[idx_v.at[j]], acc_v, sem, add=True)
        return carry

    lax.fori_loop(0, _L, gbody, 0)

    def wbody(j, carry):
        pltpu.make_async_copy(table_hbm.at[idx_v.at[0]], acc_v, sem).wait()
        return carry

    lax.fori_loop(0, _L, wbody, 0)

    pltpu.sync_copy(acc_v, out_hbm.at[pl.ds(base, _BPW), :])


@jax.jit
def _sc_pool(ids_t, table):
    mesh = plsc.VectorSubcoreMesh(core_axis_name="c", subcore_axis_name="s")
    f = pl.kernel(
        _sc_pool_body,
        out_type=jax.ShapeDtypeStruct((_B, _D), jnp.float32),
        mesh=mesh,
        scratch_types=[
            pltpu.VMEM((_L, _BPW), jnp.int32),
            pltpu.VMEM((_BPW, _D), jnp.float32),
            pltpu.SemaphoreType.DMA,
        ],
        compiler_params=pltpu.CompilerParams(use_tc_tiling_on_sc=False),
    )
    return f(ids_t, table)


def _tc_detile_body(x_ref, o_ref):
    o_ref[...] = x_ref[...].reshape(-1)


@jax.jit
def _tc_detile(ids_t):
    return pl.pallas_call(
        _tc_detile_body,
        out_shape=jax.ShapeDtypeStruct((_L * _B,), jnp.int32),
    )(ids_t)


_SLABW = 512
_NSLAB = _VOCAB // _SLABW
_SPW = _NSLAB // _NW
_TAIL = _VOCAB - _NSLAB * _SLABW


def _sc_transpose_body(tt_hbm, tail_hbm, out_hbm, slab_v, obuf_v, tbuf_v,
                       sem_in, sem_out):
    c = lax.axis_index("c")
    s = lax.axis_index("s")
    wid = s * _NC + c
    start = wid * _SPW

    iota = lax.iota(jnp.int32, 16)

    def transpose_slab(buf_idx):
        pp = jnp.full((16,), buf_idx, jnp.int32)
        for half in range(2):
            dd = iota + 16 * half

            @plsc.parallel_loop(0, _SLABW, step=1, unroll=16)
            def gbody(g):
                cc = jnp.bitwise_and(g, 15)
                v0 = g - cc
                vv = jnp.bitwise_and(iota + cc, 15) + v0
                x = plsc.load_gather(slab_v, [pp, dd, vv])
                sidx = jnp.left_shift(vv, 5) + dd
                plsc.store_scatter(obuf_v, [pp, sidx], x)

    pltpu.async_copy(tt_hbm.at[:, pl.ds(start * _SLABW, _SLABW)],
                     slab_v.at[0], sem_in)

    def body(k, carry):
        p = lax.rem(k, 2)

        @pl.when(k + 1 < _SPW)
        def _prefetch():
            pltpu.async_copy(
                tt_hbm.at[:, pl.ds((start + k + 1) * _SLABW, _SLABW)],
                slab_v.at[lax.rem(k + 1, 2)], sem_in)

        pltpu.make_async_copy(tt_hbm.at[:, pl.ds(0, _SLABW)],
                              slab_v.at[0], sem_in).wait()

        @pl.when(k >= 2)
        def _free_obuf():
            pltpu.make_async_copy(obuf_v.at[0],
                                  out_hbm.at[pl.ds(0, _SLABW * _D)],
                                  sem_out).wait()

        transpose_slab(p)
        pltpu.async_copy(
            obuf_v.at[p],
            out_hbm.at[pl.ds((start + k) * _SLABW * _D, _SLABW * _D)],
            sem_out)
        return carry

    lax.fori_loop(0, _SPW, body, 0)

    def drain(j, carry):
        pltpu.make_async_copy(obuf_v.at[0], out_hbm.at[pl.ds(0, _SLABW * _D)],
                              sem_out).wait()
        return carry

    lax.fori_loop(0, 2, drain, 0)

    @pl.when(wid == _NW - 1)
    def _tail():
        pltpu.sync_copy(tail_hbm, tbuf_v)
        pltpu.sync_copy(tbuf_v,
                        out_hbm.at[pl.ds(_NSLAB * _SLABW * _D, _TAIL * _D)])


@jax.jit
def _sc_transpose(tt, tail1d):
    mesh = plsc.VectorSubcoreMesh(core_axis_name="c", subcore_axis_name="s")
    f = pl.kernel(
        _sc_transpose_body,
        out_type=jax.ShapeDtypeStruct((_VOCAB * _D,), jnp.float32),
        mesh=mesh,
        scratch_types=[
            pltpu.VMEM((2, _D, _SLABW), jnp.float32),
            pltpu.VMEM((2, _SLABW * _D), jnp.float32),
            pltpu.VMEM((_TAIL * _D,), jnp.float32),
            pltpu.SemaphoreType.DMA,
            pltpu.SemaphoreType.DMA,
        ],
        compiler_params=pltpu.CompilerParams(
            use_tc_tiling_on_sc=True, needs_layout_passes=False),
    )
    return f(tt, tail1d).reshape(_VOCAB, _D)


def _tc_linear_body(x_ref, wt_ref, b_ref, o_ref):
    o_ref[...] = (
        jnp.dot(x_ref[...], wt_ref[...], preferred_element_type=jnp.float32)
        + b_ref[...]
    )


@jax.jit
def _tc_linear(sums, wt_scaled, b2d):
    return pl.pallas_call(
        _tc_linear_body,
        out_shape=jax.ShapeDtypeStruct((_B, 2), jnp.float32),
    )(sums, wt_scaled, b2d)


def kernel(input_ids, embedding, W, b):
    ids_t = input_ids.T.astype(jnp.int32)
    ids_lin = _tc_detile(ids_t).reshape(_L, _B)
    tail1d = embedding[_NSLAB * _SLABW:, :].reshape(-1)
    table_lin = _sc_transpose(embedding.T, tail1d)
    sums = _sc_pool(ids_lin, table_lin)
    wt_scaled = (W.T / jnp.float32(_L)).astype(jnp.float32)
    b2d = b.reshape(1, 2).astype(jnp.float32)
    return _tc_linear(sums, wt_scaled, b2d)
